# baseline (speedup 1.0000x reference)
import functools

import jax
import jax.numpy as jnp
from jax import lax
from jax.experimental import pallas as pl
from jax.experimental.pallas import tpu as pltpu
from jax.experimental.pallas import tpu_sc as plsc

_NUM_EDGE_TYPE = 16
_EMBED_DIM = 128
_N_EDGES = 1600000

_NC = 2
_NS = 16
_NW = _NC * _NS
_SB = 40
_K = 10
_GROUP = _K * _SB
_B_PER_W = _N_EDGES // _NW
_NG = _B_PER_W // _GROUP
_IRING = 4


def _emb_body(idx_hbm, table_hbm, out_hbm, idx_v, rows_v, table_v, gsem, isem, ssem):
    wid = lax.axis_index("s") * _NC + lax.axis_index("c")
    row_base = wid * _B_PER_W

    @pl.when(lax.axis_index("s") == 0)
    def _():
        pltpu.sync_copy(table_hbm, table_v)

    plsc.subcore_barrier()

    def _idx_load_start(g, slot):
        pltpu.make_async_copy(idx_hbm.at[wid, g], idx_v.at[slot], isem).start()

    def _idx_load_wait():
        pltpu.make_async_copy(idx_hbm.at[0, 0], idx_v.at[0], isem).wait()

    def _gathers_start(islot, rslot):
        for k in range(_K):
            pltpu.make_async_copy(
                table_v.at[idx_v.at[islot, k]],
                rows_v.at[rslot, pl.ds(k * _SB, _SB)],
                gsem,
            ).start()

    def _gathers_wait():
        for k in range(_K):
            pltpu.make_async_copy(
                table_v.at[idx_v.at[0, 0]],
                rows_v.at[0, pl.ds(k * _SB, _SB)],
                gsem,
            ).wait()

    def _scatter_start(g, rslot, k):
        pltpu.make_async_copy(
            rows_v.at[rslot, pl.ds(k * _SB, _SB)],
            out_hbm.at[pl.ds(row_base + g * _GROUP + k * _SB, _SB)],
            ssem,
        ).start()

    def _scatter_wait_group():
        for _ in range(_K):
            pltpu.make_async_copy(
                rows_v.at[0, pl.ds(0, _SB)], out_hbm.at[pl.ds(0, _SB)], ssem
            ).wait()

    pltpu.sync_copy(idx_hbm.at[wid, 0], idx_v.at[0])
    if _NG > 1:
        _idx_load_start(1, 1)
    _gathers_start(0, 0)

    def body(g, _):
        rslot = lax.rem(g, 2)

        @pl.when(g + 1 < _NG)
        def _():
            _idx_load_wait()

            @pl.when(g >= 1)
            def _():
                _scatter_wait_group()

            _gathers_start(lax.rem(g + 1, _IRING), 1 - rslot)

        @pl.when(g + 2 < _NG)
        def _():
            _idx_load_start(g + 2, lax.rem(g + 2, _IRING))

        for k in range(_K):
            pltpu.make_async_copy(
                table_v.at[idx_v.at[0, 0]],
                rows_v.at[0, pl.ds(k * _SB, _SB)],
                gsem,
            ).wait()
            _scatter_start(g, rslot, k)
        return 0

    lax.fori_loop(0, _NG, body, 0)

    _scatter_wait_group()
    _scatter_wait_group()


@functools.partial(
    pl.kernel,
    mesh=plsc.VectorSubcoreMesh(core_axis_name="c", subcore_axis_name="s"),
    out_type=jax.ShapeDtypeStruct((_N_EDGES, _EMBED_DIM), jnp.float32),
    scratch_types=[
        pltpu.VMEM((_IRING, _K, _SB), jnp.int32),
        pltpu.VMEM((2, _GROUP, _EMBED_DIM), jnp.float32),
        pltpu.VMEM_SHARED((_NUM_EDGE_TYPE, _EMBED_DIM), jnp.float32),
        pltpu.SemaphoreType.DMA,
        pltpu.SemaphoreType.DMA,
        pltpu.SemaphoreType.DMA,
    ],
)
def _emb(idx_hbm, table_hbm, out_hbm, idx_v, rows_v, table_v, gsem, isem, ssem):
    _emb_body(idx_hbm, table_hbm, out_hbm, idx_v, rows_v, table_v, gsem, isem, ssem)


def kernel(data, edge_type_table):
    idx4d = data.astype(jnp.int32).reshape(_NW, _NG, _K, _SB)
    return _emb(idx4d, edge_type_table)


# speedup vs baseline: 1.0216x; 1.0216x over previous
import functools

import jax
import jax.numpy as jnp
from jax import lax
from jax.experimental import pallas as pl
from jax.experimental.pallas import tpu as pltpu
from jax.experimental.pallas import tpu_sc as plsc

_NUM_EDGE_TYPE = 16
_EMBED_DIM = 128
_N_EDGES = 1600000

_NC = 2
_NS = 16
_NW = _NC * _NS
_SB = 80
_K = 5
_GROUP = _K * _SB
_B_PER_W = _N_EDGES // _NW
_NG = _B_PER_W // _GROUP
_IRING = 4


def _emb_body(idx_hbm, table_hbm, out_hbm, idx_v, rows_v, table_v, gsem, isem, ssem):
    wid = lax.axis_index("s") * _NC + lax.axis_index("c")
    row_base = wid * _B_PER_W

    @pl.when(lax.axis_index("s") == 0)
    def _():
        pltpu.sync_copy(table_hbm, table_v)

    plsc.subcore_barrier()

    def _idx_load_start(g, slot):
        pltpu.make_async_copy(idx_hbm.at[wid, g], idx_v.at[slot], isem).start()

    def _idx_load_wait():
        pltpu.make_async_copy(idx_hbm.at[0, 0], idx_v.at[0], isem).wait()

    def _gathers_start(islot, rslot):
        for k in range(_K):
            pltpu.make_async_copy(
                table_v.at[idx_v.at[islot, k]],
                rows_v.at[rslot, pl.ds(k * _SB, _SB)],
                gsem,
            ).start()

    def _gathers_wait():
        for k in range(_K):
            pltpu.make_async_copy(
                table_v.at[idx_v.at[0, 0]],
                rows_v.at[0, pl.ds(k * _SB, _SB)],
                gsem,
            ).wait()

    def _scatter_start(g, rslot, k):
        pltpu.make_async_copy(
            rows_v.at[rslot, pl.ds(k * _SB, _SB)],
            out_hbm.at[pl.ds(row_base + g * _GROUP + k * _SB, _SB)],
            ssem,
        ).start()

    def _scatter_wait_group():
        for _ in range(_K):
            pltpu.make_async_copy(
                rows_v.at[0, pl.ds(0, _SB)], out_hbm.at[pl.ds(0, _SB)], ssem
            ).wait()

    pltpu.sync_copy(idx_hbm.at[wid, 0], idx_v.at[0])
    if _NG > 1:
        _idx_load_start(1, 1)
    _gathers_start(0, 0)

    def body(g, _):
        rslot = lax.rem(g, 2)

        @pl.when(g + 1 < _NG)
        def _():
            _idx_load_wait()

            @pl.when(g >= 1)
            def _():
                _scatter_wait_group()

            _gathers_start(lax.rem(g + 1, _IRING), 1 - rslot)

        @pl.when(g + 2 < _NG)
        def _():
            _idx_load_start(g + 2, lax.rem(g + 2, _IRING))

        for k in range(_K):
            pltpu.make_async_copy(
                table_v.at[idx_v.at[0, 0]],
                rows_v.at[0, pl.ds(k * _SB, _SB)],
                gsem,
            ).wait()
            _scatter_start(g, rslot, k)
        return 0

    lax.fori_loop(0, _NG, body, 0)

    _scatter_wait_group()
    _scatter_wait_group()


@functools.partial(
    pl.kernel,
    mesh=plsc.VectorSubcoreMesh(core_axis_name="c", subcore_axis_name="s"),
    out_type=jax.ShapeDtypeStruct((_N_EDGES, _EMBED_DIM), jnp.float32),
    scratch_types=[
        pltpu.VMEM((_IRING, _K, _SB), jnp.int32),
        pltpu.VMEM((2, _GROUP, _EMBED_DIM), jnp.float32),
        pltpu.VMEM_SHARED((_NUM_EDGE_TYPE, _EMBED_DIM), jnp.float32),
        pltpu.SemaphoreType.DMA,
        pltpu.SemaphoreType.DMA,
        pltpu.SemaphoreType.DMA,
    ],
)
def _emb(idx_hbm, table_hbm, out_hbm, idx_v, rows_v, table_v, gsem, isem, ssem):
    _emb_body(idx_hbm, table_hbm, out_hbm, idx_v, rows_v, table_v, gsem, isem, ssem)


def kernel(data, edge_type_table):
    idx4d = data.astype(jnp.int32).reshape(_NW, _NG, _K, _SB)
    return _emb(idx4d, edge_type_table)
